# baseline (speedup 1.0000x reference)
import jax
import jax.numpy as jnp
from jax import lax
from jax.experimental import pallas as pl
from jax.experimental.pallas import tpu as pltpu

_B = 32
_H = 1024
_W = 1024
_N = _H * _W
_K = 838860
_KF = float(_K)
_EPS = 1e-06
_HI_MASK = -65536


def _staged_sum(a):
    s1 = jnp.sum(a.reshape(8, 128, _W), axis=0)
    s2 = jnp.sum(s1.reshape(8, 16, _W), axis=0)
    return jnp.sum(s2)


def _bf16_count_less(y, cand16):
    cb = lax.bitcast_convert_type(cand16.astype(jnp.int16), jnp.bfloat16)
    t = (y < cb).astype(jnp.bfloat16)
    for _ in range(7):
        h = t.shape[0] // 2
        t = t[:h] + t[h:]
    tf = t.astype(jnp.float32)
    return jnp.sum(tf)


def _lcl_batch_kernel(x_ref, out_ref, l_ref, y_ref):
    x = x_ref[0]
    z_row = jnp.zeros((1, _W), jnp.float32)
    z_col = jnp.zeros((_H, 1), jnp.float32)
    up = jnp.concatenate([x[1:, :], z_row], axis=0)
    down = jnp.concatenate([z_row, x[:-1, :]], axis=0)
    left = jnp.concatenate([x[:, 1:], z_col], axis=1)
    right = jnp.concatenate([z_col, x[:, :-1]], axis=1)
    L = jnp.abs(up + down + left + right - 4.0 * x)
    l_ref[...] = L
    total = _staged_sum(L)

    xi0 = lax.bitcast_convert_type(L, jnp.int32)
    y_ref[...] = lax.bitcast_convert_type(
        xi0 & _HI_MASK, jnp.float32).astype(jnp.bfloat16)

    def body_a(i, carry):
        prefix, c = carry
        cand = prefix | (jnp.int32(1) << (14 - i))
        cnt = _bf16_count_less(y_ref[...], cand)
        acc = cnt <= _KF
        return lax.select(acc, cand, prefix), lax.select(acc, cnt, c)

    hi16, c_hi = lax.fori_loop(
        0, 15, body_a, (jnp.int32(0), jnp.float32(0.0)))
    hi_shift = hi16 << 16
    k2 = _KF - c_hi

    xi = lax.bitcast_convert_type(l_ref[...], jnp.int32)
    inbin = (xi & _HI_MASK) == hi_shift
    lo14 = (xi >> 2) & jnp.int32(0x3FFF)
    mv = jnp.where(inbin, lo14, jnp.int32(0x7F00))
    y_ref[...] = lax.bitcast_convert_type(
        mv << 16, jnp.float32).astype(jnp.bfloat16)

    def body_b(i, prefix):
        cand = prefix | (jnp.int32(1) << (13 - i))
        cnt = _bf16_count_less(y_ref[...], cand)
        return lax.select(cnt <= k2, cand, prefix)

    m14 = lax.fori_loop(0, 14, body_b, jnp.int32(0))

    def body_c(i, prefix):
        cand = prefix | (jnp.int32(1) << (1 - i))
        m = (xi < cand).astype(jnp.float32)
        cnt = _staged_sum(m)
        return lax.select(cnt <= _KF, cand, prefix)

    v = lax.fori_loop(0, 2, body_c, hi_shift | (m14 << 2))

    maskf = (xi > v).astype(jnp.float32)
    edge_sum = _staged_sum(L * maskf)
    edge_cnt = _staged_sum(maskf)

    lane = lax.broadcasted_iota(jnp.int32, (1, 1, 128), 2)
    vec = jnp.where(lane == 0, edge_sum,
          jnp.where(lane == 1, edge_cnt,
          jnp.where(lane == 2, total, 0.0)))
    out_ref[...] = vec


def _combine_kernel(p_ref, out_ref):
    p = p_ref[...]
    es = jnp.sum(p[:, :, 0:1])
    ec = jnp.sum(p[:, :, 1:2])
    fs = jnp.sum(p[:, :, 2:3]) - es
    fc = jnp.float32(_B * _N) - ec
    edge_mean = jnp.where(ec > 0, es / jnp.maximum(ec, 1.0), 0.0)
    flat_mean = jnp.where(fc > 0, fs / jnp.maximum(fc, 1.0), jnp.float32(_EPS))
    result = flat_mean / (edge_mean + jnp.float32(_EPS))
    out_ref[...] = jnp.broadcast_to(result, (1, 128))


def kernel(pred):
    x = pred.reshape(_B, _H, _W)
    partials = pl.pallas_call(
        _lcl_batch_kernel,
        out_shape=jax.ShapeDtypeStruct((_B, 1, 128), jnp.float32),
        grid=(_B,),
        in_specs=[pl.BlockSpec((1, _H, _W), lambda b: (b, 0, 0))],
        out_specs=pl.BlockSpec((1, 1, 128), lambda b: (b, 0, 0)),
        scratch_shapes=[
            pltpu.VMEM((_H, _W), jnp.float32),
            pltpu.VMEM((_H, _W), jnp.bfloat16),
        ],
        compiler_params=pltpu.CompilerParams(
            dimension_semantics=("parallel",),
        ),
        name="lcl_batch",
    )(x)
    out = pl.pallas_call(
        _combine_kernel,
        out_shape=jax.ShapeDtypeStruct((1, 128), jnp.float32),
        name="lcl_combine",
    )(partials)
    return out[0, 0]


# speedup vs baseline: 2.0652x; 2.0652x over previous
import jax
import jax.numpy as jnp
from jax import lax
from jax.experimental import pallas as pl
from jax.experimental.pallas import tpu as pltpu

_B = 32
_H = 1024
_W = 1024
_N = _H * _W
_K = 838860
_KF = float(_K)
_EPS = 1e-06
_SLAB = 64


def _slab_fold(make_slab):
    acc = jnp.zeros((8, _W), jnp.float32)
    for r in range(0, _H, _SLAB):
        m = make_slab(r)
        acc = acc + jnp.sum(m.reshape(8, 8, _W), axis=0)
    return jnp.sum(acc)


def _lcl_batch_kernel(x_ref, out_ref, l_ref):
    x = x_ref[0]
    z_row = jnp.zeros((1, _W), jnp.float32)
    z_col = jnp.zeros((_H, 1), jnp.float32)
    up = jnp.concatenate([x[1:, :], z_row], axis=0)
    down = jnp.concatenate([z_row, x[:-1, :]], axis=0)
    left = jnp.concatenate([x[:, 1:], z_col], axis=1)
    right = jnp.concatenate([z_col, x[:, :-1]], axis=1)
    l_ref[...] = jnp.abs(up + down + left + right - 4.0 * x)

    total = _slab_fold(lambda r: l_ref[r:r + _SLAB, :])

    def count_less(cand):
        def slab(r):
            xi = lax.bitcast_convert_type(l_ref[r:r + _SLAB, :], jnp.int32)
            return jnp.where(xi < cand, 1.0, 0.0)
        return _slab_fold(slab)

    def body(i, prefix):
        cand = prefix | (jnp.int32(1) << (30 - i))
        return lax.select(count_less(cand) <= _KF, cand, prefix)

    v = lax.fori_loop(0, 31, body, jnp.int32(0))

    def edge_slab(r):
        lv = l_ref[r:r + _SLAB, :]
        xi = lax.bitcast_convert_type(lv, jnp.int32)
        return jnp.where(xi > v, lv, 0.0)

    def cnt_slab(r):
        xi = lax.bitcast_convert_type(l_ref[r:r + _SLAB, :], jnp.int32)
        return jnp.where(xi > v, 1.0, 0.0)

    edge_sum = _slab_fold(edge_slab)
    edge_cnt = _slab_fold(cnt_slab)

    lane = lax.broadcasted_iota(jnp.int32, (1, 1, 128), 2)
    vec = jnp.where(lane == 0, edge_sum,
          jnp.where(lane == 1, edge_cnt,
          jnp.where(lane == 2, total, 0.0)))
    out_ref[...] = vec


def _combine_kernel(p_ref, out_ref):
    p = p_ref[...]
    es = jnp.sum(p[:, :, 0:1])
    ec = jnp.sum(p[:, :, 1:2])
    fs = jnp.sum(p[:, :, 2:3]) - es
    fc = jnp.float32(_B * _N) - ec
    edge_mean = jnp.where(ec > 0, es / jnp.maximum(ec, 1.0), 0.0)
    flat_mean = jnp.where(fc > 0, fs / jnp.maximum(fc, 1.0), jnp.float32(_EPS))
    result = flat_mean / (edge_mean + jnp.float32(_EPS))
    out_ref[...] = jnp.broadcast_to(result, (1, 128))


def kernel(pred):
    x = pred.reshape(_B, _H, _W)
    partials = pl.pallas_call(
        _lcl_batch_kernel,
        out_shape=jax.ShapeDtypeStruct((_B, 1, 128), jnp.float32),
        grid=(_B,),
        in_specs=[pl.BlockSpec((1, _H, _W), lambda b: (b, 0, 0))],
        out_specs=pl.BlockSpec((1, 1, 128), lambda b: (b, 0, 0)),
        scratch_shapes=[
            pltpu.VMEM((_H, _W), jnp.float32),
        ],
        compiler_params=pltpu.CompilerParams(
            dimension_semantics=("parallel",),
        ),
        name="lcl_batch",
    )(x)
    out = pl.pallas_call(
        _combine_kernel,
        out_shape=jax.ShapeDtypeStruct((1, 128), jnp.float32),
        name="lcl_combine",
    )(partials)
    return out[0, 0]


# speedup vs baseline: 2.0727x; 1.0036x over previous
import jax
import jax.numpy as jnp
from jax import lax
from jax.experimental import pallas as pl
from jax.experimental.pallas import tpu as pltpu

_B = 32
_H = 1024
_W = 1024
_N = _H * _W
_K = 838860
_KF = float(_K)
_EPS = 1e-06
_SLAB = 64


def _fold8(m):
    return jnp.sum(m.reshape(8, 8, _W), axis=0)


def _lcl_batch_kernel(x_ref, out_ref, l_ref):
    x = x_ref[0]
    z_row = jnp.zeros((1, _W), jnp.float32)
    z_col = jnp.zeros((_H, 1), jnp.float32)
    up = jnp.concatenate([x[1:, :], z_row], axis=0)
    down = jnp.concatenate([z_row, x[:-1, :]], axis=0)
    left = jnp.concatenate([x[:, 1:], z_col], axis=1)
    right = jnp.concatenate([z_col, x[:, :-1]], axis=1)
    lap = jnp.abs(up + down + left + right - 4.0 * x)
    l_ref[...] = lap

    tacc = jnp.zeros((8, _W), jnp.float32)
    for r in range(0, _H, _SLAB):
        tacc = tacc + _fold8(lap[r:r + _SLAB, :])
    total = jnp.sum(tacc)

    def count_less(cand):
        acc = jnp.zeros((8, _W), jnp.float32)
        for r in range(0, _H, _SLAB):
            xi = lax.bitcast_convert_type(l_ref[r:r + _SLAB, :], jnp.int32)
            acc = acc + _fold8(jnp.where(xi < cand, 1.0, 0.0))
        return jnp.sum(acc)

    def body(i, prefix):
        cand = prefix | (jnp.int32(1) << (30 - i))
        return lax.select(count_less(cand) <= _KF, cand, prefix)

    v = lax.fori_loop(0, 31, body, jnp.int32(0))

    sacc = jnp.zeros((8, _W), jnp.float32)
    cacc = jnp.zeros((8, _W), jnp.float32)
    for r in range(0, _H, _SLAB):
        lv = l_ref[r:r + _SLAB, :]
        xi = lax.bitcast_convert_type(lv, jnp.int32)
        mask = xi > v
        sacc = sacc + _fold8(jnp.where(mask, lv, 0.0))
        cacc = cacc + _fold8(jnp.where(mask, 1.0, 0.0))
    edge_sum = jnp.sum(sacc)
    edge_cnt = jnp.sum(cacc)

    lane = lax.broadcasted_iota(jnp.int32, (1, 1, 128), 2)
    vec = jnp.where(lane == 0, edge_sum,
          jnp.where(lane == 1, edge_cnt,
          jnp.where(lane == 2, total, 0.0)))
    out_ref[...] = vec


def _combine_kernel(p_ref, out_ref):
    p = p_ref[...]
    es = jnp.sum(p[:, :, 0:1])
    ec = jnp.sum(p[:, :, 1:2])
    fs = jnp.sum(p[:, :, 2:3]) - es
    fc = jnp.float32(_B * _N) - ec
    edge_mean = jnp.where(ec > 0, es / jnp.maximum(ec, 1.0), 0.0)
    flat_mean = jnp.where(fc > 0, fs / jnp.maximum(fc, 1.0), jnp.float32(_EPS))
    result = flat_mean / (edge_mean + jnp.float32(_EPS))
    out_ref[...] = jnp.broadcast_to(result, (1, 128))


def kernel(pred):
    x = pred.reshape(_B, _H, _W)
    partials = pl.pallas_call(
        _lcl_batch_kernel,
        out_shape=jax.ShapeDtypeStruct((_B, 1, 128), jnp.float32),
        grid=(_B,),
        in_specs=[pl.BlockSpec((1, _H, _W), lambda b: (b, 0, 0))],
        out_specs=pl.BlockSpec((1, 1, 128), lambda b: (b, 0, 0)),
        scratch_shapes=[
            pltpu.VMEM((_H, _W), jnp.float32),
        ],
        compiler_params=pltpu.CompilerParams(
            dimension_semantics=("parallel",),
        ),
        name="lcl_batch",
    )(x)
    out = pl.pallas_call(
        _combine_kernel,
        out_shape=jax.ShapeDtypeStruct((1, 128), jnp.float32),
        name="lcl_combine",
    )(partials)
    return out[0, 0]
